# baseline (speedup 1.0000x reference)
import functools

import jax
import jax.numpy as jnp
from jax import lax
from jax.experimental import pallas as pl
from jax.experimental.pallas import tpu as pltpu
from jax.experimental.pallas import tpu_sc as plsc

N = 10000
E = 160000
F = 256
HF = 128
BLK = 1000
CH = 128
NCH = E // CH
NSUB = 16
RPS = 624
TAIL = N - RPS * NSUB

_f32 = jnp.float32
_mesh = plsc.VectorSubcoreMesh(core_axis_name="c", subcore_axis_name="s")


@functools.partial(
    pl.kernel,
    out_type=(jax.ShapeDtypeStruct((N, 16), _f32),
              jax.ShapeDtypeStruct((N, 16), _f32)),
    mesh=_mesh,
    scratch_types=[
        pltpu.VMEM((CH,), jnp.int32),
        pltpu.VMEM((CH, 16), _f32),
        pltpu.VMEM_SHARED((N, 16), _f32),
    ],
)
def _sc_degree(dst_hbm, ones_hbm, degA_hbm, degB_hbm, dstbuf, onesbuf, acc):
    c = lax.axis_index("c")
    s = lax.axis_index("s")

    def rowcopy(from_ref, to_ref):
        sl = pl.ds(s * RPS, RPS)
        pltpu.sync_copy(from_ref.at[sl], to_ref.at[sl])

        @pl.when(s == NSUB - 1)
        def _():
            tl = pl.ds(RPS * NSUB, TAIL)
            pltpu.sync_copy(from_ref.at[tl], to_ref.at[tl])

    rowcopy(ones_hbm, acc)
    pltpu.sync_copy(ones_hbm.at[pl.ds(0, CH)], onesbuf)
    plsc.subcore_barrier()
    w = c * NSUB + s
    lo = (w * NCH) // (2 * NSUB)
    hi = ((w + 1) * NCH) // (2 * NSUB)

    def chunk(j, carry):
        pltpu.sync_copy(dst_hbm.at[pl.ds(j * CH, CH)], dstbuf)
        pltpu.sync_copy(onesbuf, acc.at[dstbuf], add=True)
        return carry

    lax.fori_loop(lo, hi, chunk, 0)
    plsc.subcore_barrier()

    @pl.when(c == 0)
    def _():
        rowcopy(acc, degA_hbm)

    @pl.when(c == 1)
    def _():
        rowcopy(acc, degB_hbm)


ECH = 128
NCHS = 80
EPS = NCHS * ECH
EPAD = EPS * NSUB
NPAD = N + 16
UNR = 12
NTAIL = NCHS % UNR


@functools.partial(
    pl.kernel,
    out_type=(jax.ShapeDtypeStruct((N, HF), _f32),
              jax.ShapeDtypeStruct((N, HF), _f32)),
    mesh=_mesh,
    scratch_types=[
        pltpu.VMEM((8, ECH), jnp.int32),
        pltpu.VMEM((3, ECH, HF), _f32),
        pltpu.VMEM_SHARED((NPAD, HF), _f32),
        pltpu.SemaphoreType.DMA,
        pltpu.SemaphoreType.DMA,
        pltpu.SemaphoreType.DMA,
        pltpu.SemaphoreType.DMA,
        pltpu.SemaphoreType.DMA,
        pltpu.SemaphoreType.DMA,
        pltpu.SemaphoreType.DMA,
        pltpu.SemaphoreType.DMA,
        pltpu.SemaphoreType.DMA,
        pltpu.SemaphoreType.DMA,
        pltpu.SemaphoreType.DMA,
        pltpu.SemaphoreType.DMA,
        pltpu.SemaphoreType.DMA,
        pltpu.SemaphoreType.DMA,
    ],
)
def _sc_agg(src_hbm, dst_hbm, hL_hbm, hR_hbm, aggL_hbm, aggR_hbm,
            idxbuf, rowbuf, acc,
            semg0, semg1, semg2, sems0, sems1, sems2,
            semis0, semis1, semis2, semis3,
            semid0, semid1, semid2, semid3):
    c = lax.axis_index("c")
    s = lax.axis_index("s")
    semg = (semg0, semg1, semg2)
    sems = (sems0, sems1, sems2)
    semis = (semis0, semis1, semis2, semis3)
    semid = (semid0, semid1, semid2, semid3)

    def rowcopy(from_ref, to_ref):
        sl = pl.ds(s * RPS, RPS)
        pltpu.sync_copy(from_ref.at[sl], to_ref.at[sl])

        @pl.when(s == NSUB - 1)
        def _():
            tl = pl.ds(RPS * NSUB, TAIL)
            pltpu.sync_copy(from_ref.at[tl], to_ref.at[tl])

    def run(h_hbm, out_hbm):
        rowcopy(h_hbm, acc)

        def idx_dma(j, bi, which, base):
            hbm = src_hbm if which == 0 else dst_hbm
            sem = semis[bi] if which == 0 else semid[bi]
            return pltpu.make_async_copy(
                hbm.at[pl.ds(base + j * ECH, ECH)],
                idxbuf.at[4 * which + bi], sem)

        def gat_dma(j, br, bi):
            return pltpu.make_async_copy(
                h_hbm.at[idxbuf.at[bi]], rowbuf.at[br], semg[br])

        def sct_dma(br, bi):
            return pltpu.make_async_copy(
                rowbuf.at[br], acc.at[idxbuf.at[4 + bi]], sems[br])

        base = s * EPS
        for j in range(3):
            idx_dma(j, j, 0, base).start()
            idx_dma(j, j, 1, base).start()
        idx_dma(0, 0, 0, base).wait()
        gat_dma(0, 0, 0).start()
        idx_dma(1, 1, 0, base).wait()
        gat_dma(1, 1, 1).start()
        plsc.subcore_barrier()

        def step(j, u, tail):
            br, bi = u % 3, u % 4
            gat_dma(j, br, bi).wait()
            idx_dma(j, bi, 1, base).wait()
            sct_dma(br, bi).start(add=True)
            if tail or u > 0:
                sct_dma((br - 1) % 3, (bi - 1) % 4).wait()
            else:
                @pl.when(j >= 1)
                def _():
                    sct_dma((br - 1) % 3, (bi - 1) % 4).wait()

            if not tail or j + 3 < NCHS:
                idx_dma(j + 3, (bi + 3) % 4, 0, base).start()
                idx_dma(j + 3, (bi + 3) % 4, 1, base).start()
            if not tail or j + 2 < NCHS:
                idx_dma(j + 2, (bi + 2) % 4, 0, base).wait()
                gat_dma(j + 2, (br + 2) % 3, (bi + 2) % 4).start()

        def group(k, carry):
            for u in range(UNR):
                step(UNR * k + u, u, False)
            return carry

        lax.fori_loop(0, NCHS // UNR, group, 0)
        for u in range(NTAIL):
            step(NCHS - NTAIL + u, (NCHS - NTAIL + u) % UNR, True)
        sct_dma((NCHS - 1) % 3, (NCHS - 1) % 4).wait()
        plsc.subcore_barrier()
        rowcopy(acc, out_hbm)

    @pl.when(c == 0)
    def _():
        run(hL_hbm, aggL_hbm)

    @pl.when(c == 1)
    def _():
        run(hR_hbm, aggR_hbm)


def _dot(a, b):
    return jnp.dot(a, b, preferred_element_type=_f32,
                   precision=lax.Precision.HIGHEST)


def _tc1_body(x_ref, w_ref, dA_ref, dB_ref, hL_ref, hR_ref, dinv_ref):
    deg = dA_ref[:, 0:1] + dB_ref[:, 0:1] - 1.0
    dinv = lax.rsqrt(deg)
    h = _dot(x_ref[...], w_ref[...]) * dinv
    hL_ref[...] = h[:, :HF]
    hR_ref[...] = h[:, HF:]
    dinv_ref[...] = dinv


def _tc1(x, W1, degA, degB):
    return pl.pallas_call(
        _tc1_body,
        grid=(N // BLK,),
        in_specs=[
            pl.BlockSpec((BLK, F), lambda i: (i, 0)),
            pl.BlockSpec((F, F), lambda i: (0, 0)),
            pl.BlockSpec((BLK, 16), lambda i: (i, 0)),
            pl.BlockSpec((BLK, 16), lambda i: (i, 0)),
        ],
        out_specs=[
            pl.BlockSpec((BLK, HF), lambda i: (i, 0)),
            pl.BlockSpec((BLK, HF), lambda i: (i, 0)),
            pl.BlockSpec((BLK, 1), lambda i: (i, 0)),
        ],
        out_shape=[
            jax.ShapeDtypeStruct((N, HF), _f32),
            jax.ShapeDtypeStruct((N, HF), _f32),
            jax.ShapeDtypeStruct((N, 1), _f32),
        ],
    )(x, W1, degA, degB)


def _tcl_body(aL_ref, aR_ref, dinv_ref, b_ref, w_ref, hL_ref, hR_ref):
    dinv = dinv_ref[...]
    agg = jnp.concatenate([aL_ref[...], aR_ref[...]], axis=1)
    t = jnp.maximum(agg * dinv + b_ref[...], 0.0)
    h = _dot(t, w_ref[...]) * dinv
    hL_ref[...] = h[:, :HF]
    hR_ref[...] = h[:, HF:]


def _tcl(aL, aR, dinv, b, W):
    return pl.pallas_call(
        _tcl_body,
        grid=(N // BLK,),
        in_specs=[
            pl.BlockSpec((BLK, HF), lambda i: (i, 0)),
            pl.BlockSpec((BLK, HF), lambda i: (i, 0)),
            pl.BlockSpec((BLK, 1), lambda i: (i, 0)),
            pl.BlockSpec((1, F), lambda i: (0, 0)),
            pl.BlockSpec((F, F), lambda i: (0, 0)),
        ],
        out_specs=[
            pl.BlockSpec((BLK, HF), lambda i: (i, 0)),
            pl.BlockSpec((BLK, HF), lambda i: (i, 0)),
        ],
        out_shape=[
            jax.ShapeDtypeStruct((N, HF), _f32),
            jax.ShapeDtypeStruct((N, HF), _f32),
        ],
    )(aL, aR, dinv, b, W)


def _tcf_body(aL_ref, aR_ref, dinv_ref, b_ref, wl_ref, bl_ref, y_ref):
    dinv = dinv_ref[...]
    agg = jnp.concatenate([aL_ref[...], aR_ref[...]], axis=1)
    t = jnp.maximum(agg * dinv + b_ref[...], 0.0)
    y = _dot(t, wl_ref[...]) + bl_ref[...]
    y_ref[...] = jax.nn.sigmoid(y)


def _tcf(aL, aR, dinv, b3, Wl, bl):
    return pl.pallas_call(
        _tcf_body,
        grid=(N // BLK,),
        in_specs=[
            pl.BlockSpec((BLK, HF), lambda i: (i, 0)),
            pl.BlockSpec((BLK, HF), lambda i: (i, 0)),
            pl.BlockSpec((BLK, 1), lambda i: (i, 0)),
            pl.BlockSpec((1, F), lambda i: (0, 0)),
            pl.BlockSpec((F, 1), lambda i: (0, 0)),
            pl.BlockSpec((1, 1), lambda i: (0, 0)),
        ],
        out_specs=[pl.BlockSpec((BLK, 1), lambda i: (i, 0))],
        out_shape=[jax.ShapeDtypeStruct((N, 1), _f32)],
    )(aL, aR, dinv, b3, Wl, bl)


def kernel(x, edge_index, W1, b1, W2, b2, W3, b3, Wl, bl):
    ones16 = jnp.ones((N, 16), _f32)
    src, dst = edge_index[0], edge_index[1]
    srcp = jnp.concatenate([src, jnp.zeros((EPAD - E,), jnp.int32)])
    dstp = jnp.concatenate([dst, jnp.full((EPAD - E,), N, jnp.int32)])
    degA, degB = _sc_degree(dst, ones16)
    hL, hR, dinv = _tc1(x, W1, degA, degB)
    aL, aR = _sc_agg(srcp, dstp, hL, hR)
    hL, hR = _tcl(aL, aR, dinv, b1.reshape(1, F), W2)
    aL, aR = _sc_agg(srcp, dstp, hL, hR)
    hL, hR = _tcl(aL, aR, dinv, b2.reshape(1, F), W3)
    aL, aR = _sc_agg(srcp, dstp, hL, hR)
    (y,) = _tcf(aL, aR, dinv, b3.reshape(1, F), Wl, bl.reshape(1, 1))
    return y.reshape(-1)


# speedup vs baseline: 1.8716x; 1.8716x over previous
import functools

import jax
import jax.numpy as jnp
from jax import lax
from jax.experimental import pallas as pl
from jax.experimental.pallas import tpu as pltpu
from jax.experimental.pallas import tpu_sc as plsc

N = 10000
E = 160000
F = 256
HF = 128
BLK = 1000
CH = 128
NCH = E // CH
NSUB = 16
RPS = 624
TAIL = N - RPS * NSUB

_f32 = jnp.float32
_mesh = plsc.VectorSubcoreMesh(core_axis_name="c", subcore_axis_name="s")


@functools.partial(
    pl.kernel,
    out_type=(jax.ShapeDtypeStruct((N, 16), _f32),
              jax.ShapeDtypeStruct((N, 16), _f32)),
    mesh=_mesh,
    scratch_types=[
        pltpu.VMEM((CH,), jnp.int32),
        pltpu.VMEM((CH, 16), _f32),
        pltpu.VMEM_SHARED((N, 16), _f32),
    ],
)
def _sc_degree(dst_hbm, ones_hbm, degA_hbm, degB_hbm, dstbuf, onesbuf, acc):
    c = lax.axis_index("c")
    s = lax.axis_index("s")

    def rowcopy(from_ref, to_ref):
        sl = pl.ds(s * RPS, RPS)
        pltpu.sync_copy(from_ref.at[sl], to_ref.at[sl])

        @pl.when(s == NSUB - 1)
        def _():
            tl = pl.ds(RPS * NSUB, TAIL)
            pltpu.sync_copy(from_ref.at[tl], to_ref.at[tl])

    rowcopy(ones_hbm, acc)
    pltpu.sync_copy(ones_hbm.at[pl.ds(0, CH)], onesbuf)
    plsc.subcore_barrier()
    w = c * NSUB + s
    lo = (w * NCH) // (2 * NSUB)
    hi = ((w + 1) * NCH) // (2 * NSUB)

    def chunk(j, carry):
        pltpu.sync_copy(dst_hbm.at[pl.ds(j * CH, CH)], dstbuf)
        pltpu.sync_copy(onesbuf, acc.at[dstbuf], add=True)
        return carry

    lax.fori_loop(lo, hi, chunk, 0)
    plsc.subcore_barrier()

    @pl.when(c == 0)
    def _():
        rowcopy(acc, degA_hbm)

    @pl.when(c == 1)
    def _():
        rowcopy(acc, degB_hbm)


EPS = E // NSUB
NCHS = EPS // CH
REM = EPS - NCHS * CH


@functools.partial(
    pl.kernel,
    out_type=(jax.ShapeDtypeStruct((N, HF), _f32),
              jax.ShapeDtypeStruct((N, HF), _f32)),
    mesh=_mesh,
    scratch_types=[
        pltpu.VMEM((EPS,), jnp.int32),
        pltpu.VMEM((2, CH), jnp.int32),
        pltpu.VMEM((1, REM), jnp.int32),
        pltpu.VMEM((2, CH, HF), _f32),
        pltpu.VMEM((REM, HF), _f32),
        pltpu.VMEM_SHARED((N, HF), _f32),
        pltpu.SemaphoreType.DMA,
        pltpu.SemaphoreType.DMA,
        pltpu.SemaphoreType.DMA,
        pltpu.SemaphoreType.DMA,
        pltpu.SemaphoreType.DMA,
        pltpu.SemaphoreType.DMA,
    ],
)
def _sc_agg(src_hbm, dst_hbm, hL_hbm, hR_hbm, aggL_hbm, aggR_hbm,
            srcslab, dstbuf, dstbuf16, rowbuf, rowbuf16, acc,
            semd0, semd1, semg0, semg1, sems0, sems1):
    c = lax.axis_index("c")
    s = lax.axis_index("s")
    semd = (semd0, semd1)
    semg = (semg0, semg1)
    sems = (sems0, sems1)

    def rowcopy(from_ref, to_ref):
        sl = pl.ds(s * RPS, RPS)
        pltpu.sync_copy(from_ref.at[sl], to_ref.at[sl])

        @pl.when(s == NSUB - 1)
        def _():
            tl = pl.ds(RPS * NSUB, TAIL)
            pltpu.sync_copy(from_ref.at[tl], to_ref.at[tl])

    def run(h_hbm, out_hbm):
        pltpu.sync_copy(src_hbm.at[pl.ds(s * EPS, EPS)], srcslab)
        rowcopy(h_hbm, acc)

        def dst_dma(j, b):
            return pltpu.make_async_copy(
                dst_hbm.at[pl.ds(s * EPS + j * CH, CH)], dstbuf.at[b], semd[b])

        def gat_dma(j, b):
            return pltpu.make_async_copy(
                h_hbm.at[srcslab.at[pl.ds(j * CH, CH)]], rowbuf.at[b], semg[b])

        def sct_dma(b):
            return pltpu.make_async_copy(rowbuf.at[b], acc.at[dstbuf.at[b]],
                                         sems[b])

        gat_dma(0, 0).start()
        dst_dma(0, 0).start()
        plsc.subcore_barrier()

        def pair(k, carry):
            for b in (0, 1):
                j = 2 * k + b
                gat_dma(j, b).wait()
                dst_dma(j, b).wait()
                sct_dma(b).start(add=True)

                @pl.when(j >= 1)
                def _():
                    sct_dma(1 - b).wait()

                @pl.when(j + 1 < NCHS)
                def _():
                    gat_dma(j + 1, 1 - b).start()
                    dst_dma(j + 1, 1 - b).start()

            return carry

        lax.fori_loop(0, NCHS // 2, pair, 0)
        sct_dma((NCHS - 1) % 2).wait()
        pltpu.sync_copy(dst_hbm.at[pl.ds(s * EPS + NCHS * CH, REM)],
                        dstbuf16.at[0])
        pltpu.async_copy(h_hbm.at[srcslab.at[pl.ds(NCHS * CH, REM)]],
                         rowbuf16, semg0).wait()
        pltpu.sync_copy(rowbuf16, acc.at[dstbuf16.at[0]], add=True)
        plsc.subcore_barrier()
        rowcopy(acc, out_hbm)

    @pl.when(c == 0)
    def _():
        run(hL_hbm, aggL_hbm)

    @pl.when(c == 1)
    def _():
        run(hR_hbm, aggR_hbm)


def _dot(a, b):
    return jnp.dot(a, b, preferred_element_type=_f32,
                   precision=lax.Precision.HIGHEST)


def _tcm_body(x_ref, w_ref, u_ref):
    u_ref[...] = _dot(x_ref[...], w_ref[...])


def _tcm(x, W1):
    return pl.pallas_call(
        _tcm_body,
        grid=(N // BLK,),
        in_specs=[
            pl.BlockSpec((BLK, F), lambda i: (i, 0)),
            pl.BlockSpec((F, F), lambda i: (0, 0)),
        ],
        out_specs=pl.BlockSpec((BLK, F), lambda i: (i, 0)),
        out_shape=jax.ShapeDtypeStruct((N, F), _f32),
    )(x, W1)


def _tc1_body(u_ref, dA_ref, dB_ref, hL_ref, hR_ref, dinv_ref):
    deg = dA_ref[:, 0:1] + dB_ref[:, 0:1] - 1.0
    dinv = lax.rsqrt(deg)
    h = u_ref[...] * dinv
    hL_ref[...] = h[:, :HF]
    hR_ref[...] = h[:, HF:]
    dinv_ref[...] = dinv


def _tc1(u, degA, degB):
    return pl.pallas_call(
        _tc1_body,
        grid=(N // BLK,),
        in_specs=[
            pl.BlockSpec((BLK, F), lambda i: (i, 0)),
            pl.BlockSpec((BLK, 16), lambda i: (i, 0)),
            pl.BlockSpec((BLK, 16), lambda i: (i, 0)),
        ],
        out_specs=[
            pl.BlockSpec((BLK, HF), lambda i: (i, 0)),
            pl.BlockSpec((BLK, HF), lambda i: (i, 0)),
            pl.BlockSpec((BLK, 1), lambda i: (i, 0)),
        ],
        out_shape=[
            jax.ShapeDtypeStruct((N, HF), _f32),
            jax.ShapeDtypeStruct((N, HF), _f32),
            jax.ShapeDtypeStruct((N, 1), _f32),
        ],
    )(u, degA, degB)


def _tcl_body(aL_ref, aR_ref, dinv_ref, b_ref, w_ref, hL_ref, hR_ref):
    dinv = dinv_ref[...]
    agg = jnp.concatenate([aL_ref[...], aR_ref[...]], axis=1)
    t = jnp.maximum(agg * dinv + b_ref[...], 0.0)
    h = _dot(t, w_ref[...]) * dinv
    hL_ref[...] = h[:, :HF]
    hR_ref[...] = h[:, HF:]


def _tcl(aL, aR, dinv, b, W):
    return pl.pallas_call(
        _tcl_body,
        grid=(N // BLK,),
        in_specs=[
            pl.BlockSpec((BLK, HF), lambda i: (i, 0)),
            pl.BlockSpec((BLK, HF), lambda i: (i, 0)),
            pl.BlockSpec((BLK, 1), lambda i: (i, 0)),
            pl.BlockSpec((1, F), lambda i: (0, 0)),
            pl.BlockSpec((F, F), lambda i: (0, 0)),
        ],
        out_specs=[
            pl.BlockSpec((BLK, HF), lambda i: (i, 0)),
            pl.BlockSpec((BLK, HF), lambda i: (i, 0)),
        ],
        out_shape=[
            jax.ShapeDtypeStruct((N, HF), _f32),
            jax.ShapeDtypeStruct((N, HF), _f32),
        ],
    )(aL, aR, dinv, b, W)


def _tcf_body(aL_ref, aR_ref, dinv_ref, b_ref, wl_ref, bl_ref, y_ref):
    dinv = dinv_ref[...]
    agg = jnp.concatenate([aL_ref[...], aR_ref[...]], axis=1)
    t = jnp.maximum(agg * dinv + b_ref[...], 0.0)
    y = _dot(t, wl_ref[...]) + bl_ref[...]
    y_ref[...] = jax.nn.sigmoid(y)


def _tcf(aL, aR, dinv, b3, Wl, bl):
    return pl.pallas_call(
        _tcf_body,
        grid=(N // BLK,),
        in_specs=[
            pl.BlockSpec((BLK, HF), lambda i: (i, 0)),
            pl.BlockSpec((BLK, HF), lambda i: (i, 0)),
            pl.BlockSpec((BLK, 1), lambda i: (i, 0)),
            pl.BlockSpec((1, F), lambda i: (0, 0)),
            pl.BlockSpec((F, 1), lambda i: (0, 0)),
            pl.BlockSpec((1, 1), lambda i: (0, 0)),
        ],
        out_specs=[pl.BlockSpec((BLK, 1), lambda i: (i, 0))],
        out_shape=[jax.ShapeDtypeStruct((N, 1), _f32)],
    )(aL, aR, dinv, b3, Wl, bl)


def kernel(x, edge_index, W1, b1, W2, b2, W3, b3, Wl, bl):
    ones16 = jnp.ones((N, 16), _f32)
    src, dst = edge_index[0], edge_index[1]
    u = _tcm(x, W1)
    degA, degB = _sc_degree(dst, ones16)
    hL, hR, dinv = _tc1(u, degA, degB)
    aL, aR = _sc_agg(src, dst, hL, hR)
    hL, hR = _tcl(aL, aR, dinv, b1.reshape(1, F), W2)
    aL, aR = _sc_agg(src, dst, hL, hR)
    hL, hR = _tcl(aL, aR, dinv, b2.reshape(1, F), W3)
    aL, aR = _sc_agg(src, dst, hL, hR)
    (y,) = _tcf(aL, aR, dinv, b3.reshape(1, F), Wl, bl.reshape(1, 1))
    return y.reshape(-1)


# speedup vs baseline: 1.8967x; 1.0134x over previous
import functools

import jax
import jax.numpy as jnp
from jax import lax
from jax.experimental import pallas as pl
from jax.experimental.pallas import tpu as pltpu
from jax.experimental.pallas import tpu_sc as plsc

N = 10000
E = 160000
F = 256
HF = 128
BLK = 1000
CH = 128
NCH = E // CH
NSUB = 16
RPS = 624
TAIL = N - RPS * NSUB

_f32 = jnp.float32
_mesh = plsc.VectorSubcoreMesh(core_axis_name="c", subcore_axis_name="s")


DSLAB = E // (2 * NSUB)
DNCH = DSLAB // CH
DREM = DSLAB - DNCH * CH


@functools.partial(
    pl.kernel,
    out_type=(jax.ShapeDtypeStruct((N, 16), _f32),
              jax.ShapeDtypeStruct((N, 16), _f32)),
    mesh=_mesh,
    scratch_types=[
        pltpu.VMEM((2, CH), jnp.int32),
        pltpu.VMEM((1, DREM), jnp.int32),
        pltpu.VMEM((CH, 16), _f32),
        pltpu.VMEM_SHARED((N, 16), _f32),
        pltpu.SemaphoreType.DMA,
        pltpu.SemaphoreType.DMA,
        pltpu.SemaphoreType.DMA,
        pltpu.SemaphoreType.DMA,
    ],
)
def _sc_degree(dst_hbm, ones_hbm, degA_hbm, degB_hbm, dstbuf, dstbuf8,
               onesbuf, acc, semd0, semd1, sems0, sems1):
    c = lax.axis_index("c")
    s = lax.axis_index("s")
    semd = (semd0, semd1)
    sems = (sems0, sems1)

    def rowcopy(from_ref, to_ref):
        sl = pl.ds(s * RPS, RPS)
        pltpu.sync_copy(from_ref.at[sl], to_ref.at[sl])

        @pl.when(s == NSUB - 1)
        def _():
            tl = pl.ds(RPS * NSUB, TAIL)
            pltpu.sync_copy(from_ref.at[tl], to_ref.at[tl])

    rowcopy(ones_hbm, acc)
    pltpu.sync_copy(ones_hbm.at[pl.ds(0, CH)], onesbuf)
    base = (c * NSUB + s) * DSLAB

    def dst_dma(j, b):
        return pltpu.make_async_copy(
            dst_hbm.at[pl.ds(base + j * CH, CH)], dstbuf.at[b], semd[b])

    def sct_dma(b):
        return pltpu.make_async_copy(onesbuf, acc.at[dstbuf.at[b]], sems[b])

    dst_dma(0, 0).start()
    plsc.subcore_barrier()

    def pair(k, carry):
        for b in (0, 1):
            j = 2 * k + b
            dst_dma(j, b).wait()
            sct_dma(b).start(add=True)

            @pl.when(j >= 1)
            def _():
                sct_dma(1 - b).wait()

            @pl.when(j + 1 < DNCH)
            def _():
                dst_dma(j + 1, 1 - b).start()

        return carry

    lax.fori_loop(0, DNCH // 2, pair, 0)
    dst_dma(DNCH - 1, 0).wait()
    sct_dma(0).start(add=True)
    sct_dma(1).wait()
    sct_dma(0).wait()
    pltpu.sync_copy(dst_hbm.at[pl.ds(base + DNCH * CH, DREM)], dstbuf8.at[0])
    pltpu.sync_copy(onesbuf.at[pl.ds(0, DREM)], acc.at[dstbuf8.at[0]],
                    add=True)
    plsc.subcore_barrier()

    @pl.when(c == 0)
    def _():
        rowcopy(acc, degA_hbm)

    @pl.when(c == 1)
    def _():
        rowcopy(acc, degB_hbm)


EPS = E // NSUB
NCHS = EPS // CH
REM = EPS - NCHS * CH


@functools.partial(
    pl.kernel,
    out_type=(jax.ShapeDtypeStruct((N, HF), _f32),
              jax.ShapeDtypeStruct((N, HF), _f32)),
    mesh=_mesh,
    scratch_types=[
        pltpu.VMEM((EPS,), jnp.int32),
        pltpu.VMEM((2, CH), jnp.int32),
        pltpu.VMEM((1, REM), jnp.int32),
        pltpu.VMEM((2, CH, HF), _f32),
        pltpu.VMEM((REM, HF), _f32),
        pltpu.VMEM_SHARED((N, HF), _f32),
        pltpu.SemaphoreType.DMA,
        pltpu.SemaphoreType.DMA,
        pltpu.SemaphoreType.DMA,
        pltpu.SemaphoreType.DMA,
        pltpu.SemaphoreType.DMA,
        pltpu.SemaphoreType.DMA,
    ],
)
def _sc_agg(src_hbm, dst_hbm, hL_hbm, hR_hbm, aggL_hbm, aggR_hbm,
            srcslab, dstbuf, dstbuf16, rowbuf, rowbuf16, acc,
            semd0, semd1, semg0, semg1, sems0, sems1):
    c = lax.axis_index("c")
    s = lax.axis_index("s")
    semd = (semd0, semd1)
    semg = (semg0, semg1)
    sems = (sems0, sems1)

    def rowcopy(from_ref, to_ref):
        sl = pl.ds(s * RPS, RPS)
        pltpu.sync_copy(from_ref.at[sl], to_ref.at[sl])

        @pl.when(s == NSUB - 1)
        def _():
            tl = pl.ds(RPS * NSUB, TAIL)
            pltpu.sync_copy(from_ref.at[tl], to_ref.at[tl])

    def run(h_hbm, out_hbm):
        pltpu.sync_copy(src_hbm.at[pl.ds(s * EPS, EPS)], srcslab)
        rowcopy(h_hbm, acc)

        def dst_dma(j, b):
            return pltpu.make_async_copy(
                dst_hbm.at[pl.ds(s * EPS + j * CH, CH)], dstbuf.at[b], semd[b])

        def gat_dma(j, b):
            return pltpu.make_async_copy(
                h_hbm.at[srcslab.at[pl.ds(j * CH, CH)]], rowbuf.at[b], semg[b])

        def sct_dma(b):
            return pltpu.make_async_copy(rowbuf.at[b], acc.at[dstbuf.at[b]],
                                         sems[b])

        gat_dma(0, 0).start()
        dst_dma(0, 0).start()
        plsc.subcore_barrier()

        def pair(k, carry):
            for b in (0, 1):
                j = 2 * k + b
                gat_dma(j, b).wait()
                dst_dma(j, b).wait()
                sct_dma(b).start(add=True)

                @pl.when(j >= 1)
                def _():
                    sct_dma(1 - b).wait()

                @pl.when(j + 1 < NCHS)
                def _():
                    gat_dma(j + 1, 1 - b).start()
                    dst_dma(j + 1, 1 - b).start()

            return carry

        lax.fori_loop(0, NCHS // 2, pair, 0)
        sct_dma((NCHS - 1) % 2).wait()
        pltpu.sync_copy(dst_hbm.at[pl.ds(s * EPS + NCHS * CH, REM)],
                        dstbuf16.at[0])
        pltpu.async_copy(h_hbm.at[srcslab.at[pl.ds(NCHS * CH, REM)]],
                         rowbuf16, semg0).wait()
        pltpu.sync_copy(rowbuf16, acc.at[dstbuf16.at[0]], add=True)
        plsc.subcore_barrier()
        rowcopy(acc, out_hbm)

    @pl.when(c == 0)
    def _():
        run(hL_hbm, aggL_hbm)

    @pl.when(c == 1)
    def _():
        run(hR_hbm, aggR_hbm)


def _dot(a, b):
    return jnp.dot(a, b, preferred_element_type=_f32,
                   precision=lax.Precision.HIGHEST)


def _tcm_body(x_ref, w_ref, u_ref):
    u_ref[...] = _dot(x_ref[...], w_ref[...])


def _tcm(x, W1):
    return pl.pallas_call(
        _tcm_body,
        grid=(N // BLK,),
        in_specs=[
            pl.BlockSpec((BLK, F), lambda i: (i, 0)),
            pl.BlockSpec((F, F), lambda i: (0, 0)),
        ],
        out_specs=pl.BlockSpec((BLK, F), lambda i: (i, 0)),
        out_shape=jax.ShapeDtypeStruct((N, F), _f32),
    )(x, W1)


def _tc1_body(u_ref, dA_ref, dB_ref, hL_ref, hR_ref, dinv_ref):
    deg = dA_ref[:, 0:1] + dB_ref[:, 0:1] - 1.0
    dinv = lax.rsqrt(deg)
    h = u_ref[...] * dinv
    hL_ref[...] = h[:, :HF]
    hR_ref[...] = h[:, HF:]
    dinv_ref[...] = dinv


def _tc1(u, degA, degB):
    return pl.pallas_call(
        _tc1_body,
        grid=(N // BLK,),
        in_specs=[
            pl.BlockSpec((BLK, F), lambda i: (i, 0)),
            pl.BlockSpec((BLK, 16), lambda i: (i, 0)),
            pl.BlockSpec((BLK, 16), lambda i: (i, 0)),
        ],
        out_specs=[
            pl.BlockSpec((BLK, HF), lambda i: (i, 0)),
            pl.BlockSpec((BLK, HF), lambda i: (i, 0)),
            pl.BlockSpec((BLK, 1), lambda i: (i, 0)),
        ],
        out_shape=[
            jax.ShapeDtypeStruct((N, HF), _f32),
            jax.ShapeDtypeStruct((N, HF), _f32),
            jax.ShapeDtypeStruct((N, 1), _f32),
        ],
    )(u, degA, degB)


def _tcl_body(aL_ref, aR_ref, dinv_ref, b_ref, w_ref, hL_ref, hR_ref):
    dinv = dinv_ref[...]
    agg = jnp.concatenate([aL_ref[...], aR_ref[...]], axis=1)
    t = jnp.maximum(agg * dinv + b_ref[...], 0.0)
    h = _dot(t, w_ref[...]) * dinv
    hL_ref[...] = h[:, :HF]
    hR_ref[...] = h[:, HF:]


def _tcl(aL, aR, dinv, b, W):
    return pl.pallas_call(
        _tcl_body,
        grid=(N // BLK,),
        in_specs=[
            pl.BlockSpec((BLK, HF), lambda i: (i, 0)),
            pl.BlockSpec((BLK, HF), lambda i: (i, 0)),
            pl.BlockSpec((BLK, 1), lambda i: (i, 0)),
            pl.BlockSpec((1, F), lambda i: (0, 0)),
            pl.BlockSpec((F, F), lambda i: (0, 0)),
        ],
        out_specs=[
            pl.BlockSpec((BLK, HF), lambda i: (i, 0)),
            pl.BlockSpec((BLK, HF), lambda i: (i, 0)),
        ],
        out_shape=[
            jax.ShapeDtypeStruct((N, HF), _f32),
            jax.ShapeDtypeStruct((N, HF), _f32),
        ],
    )(aL, aR, dinv, b, W)


def _tcf_body(aL_ref, aR_ref, dinv_ref, b_ref, wl_ref, bl_ref, y_ref):
    dinv = dinv_ref[...]
    agg = jnp.concatenate([aL_ref[...], aR_ref[...]], axis=1)
    t = jnp.maximum(agg * dinv + b_ref[...], 0.0)
    y = _dot(t, wl_ref[...]) + bl_ref[...]
    y_ref[...] = jax.nn.sigmoid(y)


def _tcf(aL, aR, dinv, b3, Wl, bl):
    return pl.pallas_call(
        _tcf_body,
        grid=(N // BLK,),
        in_specs=[
            pl.BlockSpec((BLK, HF), lambda i: (i, 0)),
            pl.BlockSpec((BLK, HF), lambda i: (i, 0)),
            pl.BlockSpec((BLK, 1), lambda i: (i, 0)),
            pl.BlockSpec((1, F), lambda i: (0, 0)),
            pl.BlockSpec((F, 1), lambda i: (0, 0)),
            pl.BlockSpec((1, 1), lambda i: (0, 0)),
        ],
        out_specs=[pl.BlockSpec((BLK, 1), lambda i: (i, 0))],
        out_shape=[jax.ShapeDtypeStruct((N, 1), _f32)],
    )(aL, aR, dinv, b3, Wl, bl)


def kernel(x, edge_index, W1, b1, W2, b2, W3, b3, Wl, bl):
    ones16 = jnp.ones((N, 16), _f32)
    src, dst = edge_index[0], edge_index[1]
    u = _tcm(x, W1)
    degA, degB = _sc_degree(dst, ones16)
    hL, hR, dinv = _tc1(u, degA, degB)
    aL, aR = _sc_agg(src, dst, hL, hR)
    hL, hR = _tcl(aL, aR, dinv, b1.reshape(1, F), W2)
    aL, aR = _sc_agg(src, dst, hL, hR)
    hL, hR = _tcl(aL, aR, dinv, b2.reshape(1, F), W3)
    aL, aR = _sc_agg(src, dst, hL, hR)
    (y,) = _tcf(aL, aR, dinv, b3.reshape(1, F), Wl, bl.reshape(1, 1))
    return y.reshape(-1)
